# Initial kernel scaffold; baseline (speedup 1.0000x reference)
import jax
import jax.numpy as jnp
from jax.experimental import pallas as pl


def kernel(mask, query, key, value, scale, region_graph):
    raise NotImplementedError("write your pallas kernel here")


# speedup vs baseline: 791.9547x; 791.9547x over previous
import functools

import jax
import jax.numpy as jnp
from jax import lax
from jax.experimental import pallas as pl
from jax.experimental.pallas import tpu as pltpu
from jax.experimental.pallas import tpu_sc as plsc

NC = 2
NS = 16
LANES = 16
BLK = 128


def _sc_attention(kT, vT, q3, idx4, scale_vec, *, nh, hd, n_vox, topk):
    nw = NC * NS
    rows_total = nh * n_vox
    rows_per_w = rows_total // nw
    n_blocks = rows_per_w // BLK
    quarters = n_vox // rows_per_w

    mesh = plsc.VectorSubcoreMesh(
        core_axis_name="c", subcore_axis_name="s",
        num_cores=NC, num_subcores=NS)

    @functools.partial(
        pl.kernel,
        out_type=[
            jax.ShapeDtypeStruct((nh, hd, n_vox), jnp.float32),
            jax.ShapeDtypeStruct((nh, n_vox, topk), jnp.float32),
        ],
        mesh=mesh,
        compiler_params=pltpu.CompilerParams(
            needs_layout_passes=False, use_tc_tiling_on_sc=False),
        scratch_types=[
            pltpu.VMEM((hd, BLK), jnp.float32),
            pltpu.VMEM((topk, BLK), jnp.int32),
            pltpu.VMEM((topk * BLK, hd), jnp.float32),
            pltpu.VMEM((topk * BLK, hd), jnp.float32),
            pltpu.VMEM((hd, BLK), jnp.float32),
            pltpu.VMEM((BLK, topk), jnp.float32),
            pltpu.VMEM((LANES,), jnp.float32),
            pltpu.SemaphoreType.DMA,
        ],
    )
    def attend(kT_hbm, vT_hbm, q_hbm, idx_hbm, scale_hbm,
               out_hbm, attn_hbm,
               qv, idxv, kg, vg, outv, attnv, scalev, sem):
        cid = lax.axis_index("c")
        sid = lax.axis_index("s")
        wid = sid * NC + cid
        h = wid // quarters
        base_n = (wid % quarters) * rows_per_w

        pltpu.sync_copy(scale_hbm, scalev)
        iota = lax.iota(jnp.int32, LANES)

        def block(i, carry):
            n0 = base_n + i * BLK
            pltpu.sync_copy(q_hbm.at[h, :, pl.ds(n0, BLK)], qv)
            pltpu.sync_copy(idx_hbm.at[h, :, pl.ds(n0, BLK)], idxv)
            handles = []
            for t in range(topk):
                handles.append(pltpu.async_copy(
                    kT_hbm.at[idxv.at[t]], kg.at[pl.ds(t * BLK, BLK)], sem))
                handles.append(pltpu.async_copy(
                    vT_hbm.at[idxv.at[t]], vg.at[pl.ds(t * BLK, BLK)], sem))
            for hnd in handles:
                hnd.wait()

            sc = scalev[...]

            def comp(j, carry2):
                c0 = j * LANES
                qvecs = [qv[d, pl.ds(c0, LANES)] for d in range(hd)]
                rowsl = [iota + (t * BLK) + c0 for t in range(topk)]
                svecs = []
                for t in range(topk):
                    acc = qvecs[0] * plsc.load_gather(
                        kg, [rowsl[t], jnp.zeros((LANES,), jnp.int32)])
                    for d in range(1, hd):
                        acc = acc + qvecs[d] * plsc.load_gather(
                            kg, [rowsl[t], jnp.full((LANES,), d, jnp.int32)])
                    svecs.append(acc * sc)
                m = jnp.maximum(jnp.maximum(svecs[0], svecs[1]),
                                jnp.maximum(svecs[2], svecs[3]))
                evecs = [jnp.exp(s - m) for s in svecs]
                denom = evecs[0] + evecs[1] + evecs[2] + evecs[3]
                pvecs = [e / denom for e in evecs]
                for t in range(topk):
                    plsc.store_scatter(
                        attnv, [iota + c0, jnp.full((LANES,), t, jnp.int32)],
                        pvecs[t])
                for d in range(hd):
                    dcol = jnp.full((LANES,), d, jnp.int32)
                    acc = pvecs[0] * plsc.load_gather(vg, [rowsl[0], dcol])
                    for t in range(1, topk):
                        acc = acc + pvecs[t] * plsc.load_gather(
                            vg, [rowsl[t], dcol])
                    outv[d, pl.ds(c0, LANES)] = acc
                return carry2

            lax.fori_loop(0, BLK // LANES, comp, 0)

            pltpu.sync_copy(outv, out_hbm.at[h, :, pl.ds(n0, BLK)])
            pltpu.sync_copy(attnv, attn_hbm.at[h, pl.ds(n0, BLK), :])
            return carry

        lax.fori_loop(0, n_blocks, block, 0)

    return attend(kT, vT, q3, idx4, scale_vec)


def kernel(mask, query, key, value, scale, region_graph):
    del mask
    b, ch, hh, ww, dd = query.shape
    _, nh, n_vox, topk = region_graph.shape
    hd = ch // nh
    assert b == 1 and n_vox == hh * ww * dd

    q3 = query.reshape(nh, hd, n_vox)
    kT = key.reshape(nh, hd, n_vox).transpose(0, 2, 1).reshape(nh * n_vox, hd)
    vT = value.reshape(nh, hd, n_vox).transpose(0, 2, 1).reshape(nh * n_vox, hd)
    idx4 = region_graph.reshape(nh, n_vox, topk).transpose(0, 2, 1)
    idx4 = (idx4 + (jnp.arange(nh, dtype=jnp.int32) * n_vox)[:, None, None]
            ).astype(jnp.int32)
    scale_vec = jnp.broadcast_to(scale.astype(jnp.float32), (LANES,))

    out3, attn3 = _sc_attention(kT, vT, q3, idx4, scale_vec,
                                nh=nh, hd=hd, n_vox=n_vox, topk=topk)
    out = out3.reshape(b, ch, hh, ww, dd)
    attn = attn3.reshape(b, nh, n_vox, 1, topk)
    return out, attn


# speedup vs baseline: 1387.3857x; 1.7518x over previous
import functools

import jax
import jax.numpy as jnp
from jax import lax
from jax.experimental import pallas as pl
from jax.experimental.pallas import tpu as pltpu
from jax.experimental.pallas import tpu_sc as plsc

NC = 2
NS = 16
LANES = 16
BLK = 128


def _sc_attention(kT, q3, idx4, scale_vec, *, nh, hd, n_vox, topk):
    nw = NC * NS
    rows_total = nh * n_vox
    rows_per_w = rows_total // nw
    n_blocks = rows_per_w // BLK
    quarters = n_vox // rows_per_w

    mesh = plsc.VectorSubcoreMesh(
        core_axis_name="c", subcore_axis_name="s",
        num_cores=NC, num_subcores=NS)

    @functools.partial(
        pl.kernel,
        out_type=[
            jax.ShapeDtypeStruct((nh, hd, n_vox), jnp.float32),
            jax.ShapeDtypeStruct((nh, n_vox, topk), jnp.float32),
        ],
        mesh=mesh,
        compiler_params=pltpu.CompilerParams(
            needs_layout_passes=False, use_tc_tiling_on_sc=False),
        scratch_types=[
            pltpu.VMEM((hd, BLK), jnp.float32),
            pltpu.VMEM((topk, BLK), jnp.int32),
            pltpu.VMEM((topk * BLK, 2 * hd), jnp.float32),
            pltpu.VMEM((hd, BLK), jnp.float32),
            pltpu.VMEM((BLK, topk), jnp.float32),
            pltpu.VMEM((LANES,), jnp.float32),
            pltpu.SemaphoreType.DMA,
        ],
    )
    def attend(kvT_hbm, q_hbm, idx_hbm, scale_hbm,
               out_hbm, attn_hbm,
               qv, idxv, kvg, outv, attnv, scalev, sem):
        cid = lax.axis_index("c")
        sid = lax.axis_index("s")
        wid = sid * NC + cid
        h = wid // quarters
        base_n = (wid % quarters) * rows_per_w

        pltpu.sync_copy(scale_hbm, scalev)
        iota = lax.iota(jnp.int32, LANES)

        def block(i, carry):
            n0 = base_n + i * BLK
            pltpu.sync_copy(q_hbm.at[h, :, pl.ds(n0, BLK)], qv)
            pltpu.sync_copy(idx_hbm.at[h, :, pl.ds(n0, BLK)], idxv)
            handles = []
            for t in range(topk):
                handles.append(pltpu.async_copy(
                    kvT_hbm.at[idxv.at[t]],
                    kvg.at[pl.ds(t * BLK, BLK)], sem))
            for hnd in handles:
                hnd.wait()

            sc = scalev[...]

            def comp(j, carry2):
                c0 = j * LANES
                ccol = iota + c0
                rowsl = [ccol + (t * BLK) for t in range(topk)]
                svecs = [None] * topk
                for i in range(hd):
                    dvec = (iota + i) & (hd - 1)
                    qg = plsc.load_gather(qv, [dvec, ccol])
                    for t in range(topk):
                        kgv = plsc.load_gather(kvg, [rowsl[t], dvec])
                        svecs[t] = (qg * kgv if svecs[t] is None
                                    else svecs[t] + qg * kgv)
                svecs = [s * sc for s in svecs]
                m = jnp.maximum(jnp.maximum(svecs[0], svecs[1]),
                                jnp.maximum(svecs[2], svecs[3]))
                evecs = [jnp.exp(s - m) for s in svecs]
                denom = evecs[0] + evecs[1] + evecs[2] + evecs[3]
                pvecs = [e / denom for e in evecs]
                for t in range(topk):
                    plsc.store_scatter(
                        attnv, [ccol, jnp.full((LANES,), t, jnp.int32)],
                        pvecs[t])
                for i in range(hd):
                    dvec = (iota + i) & (hd - 1)
                    acc = pvecs[0] * plsc.load_gather(
                        kvg, [rowsl[0], dvec + hd])
                    for t in range(1, topk):
                        acc = acc + pvecs[t] * plsc.load_gather(
                            kvg, [rowsl[t], dvec + hd])
                    plsc.store_scatter(outv, [dvec, ccol], acc)
                return carry2

            lax.fori_loop(0, BLK // LANES, comp, 0)

            pltpu.sync_copy(outv, out_hbm.at[h, :, pl.ds(n0, BLK)])
            pltpu.sync_copy(attnv, attn_hbm.at[h, pl.ds(n0, BLK), :])
            return carry

        lax.fori_loop(0, n_blocks, block, 0)

    return attend(kT, q3, idx4, scale_vec)


def kernel(mask, query, key, value, scale, region_graph):
    del mask
    b, ch, hh, ww, dd = query.shape
    _, nh, n_vox, topk = region_graph.shape
    hd = ch // nh
    assert b == 1 and n_vox == hh * ww * dd

    q3 = query.reshape(nh, hd, n_vox)
    kv = jnp.concatenate(
        [key.reshape(nh, hd, n_vox), value.reshape(nh, hd, n_vox)], axis=1)
    kT = kv.transpose(0, 2, 1).reshape(nh * n_vox, 2 * hd)
    idx4 = region_graph.reshape(nh, n_vox, topk).transpose(0, 2, 1)
    idx4 = (idx4 + (jnp.arange(nh, dtype=jnp.int32) * n_vox)[:, None, None]
            ).astype(jnp.int32)
    scale_vec = jnp.broadcast_to(scale.astype(jnp.float32), (LANES,))

    out3, attn3 = _sc_attention(kT, q3, idx4, scale_vec,
                                nh=nh, hd=hd, n_vox=n_vox, topk=topk)
    out = out3.reshape(b, ch, hh, ww, dd)
    attn = attn3.reshape(b, nh, n_vox, 1, topk)
    return out, attn
